# baseline (device time: 31003 ns/iter reference)
import jax
import jax.numpy as jnp
from jax import lax
from jax.experimental import pallas as pl
from jax.experimental.pallas import tpu as pltpu

N_DEV = 4
BLK = 512
N = 2048


def kernel(x, w_mat):
    def body(x_hbm, w_hbm, out_hbm, xf32_ref, xbf_ref, wf32_ref, wbf_ref,
             comm_ref, obuf_ref, x_sem, w_sems, o_sems, send_sems, recv_sems):
        my = lax.axis_index("i")

        x_dma = pltpu.make_async_copy(x_hbm, xf32_ref, x_sem)
        x_dma.start()
        w_dmas = []
        for j in range(N_DEV):
            dma = pltpu.make_async_copy(
                w_hbm.at[pl.ds(j * BLK, BLK), :],
                wf32_ref.at[pl.ds(j * BLK, BLK), :],
                w_sems.at[j],
            )
            dma.start()
            w_dmas.append(dma)

        barrier_sem = pltpu.get_barrier_semaphore()
        for off in range(1, N_DEV):
            peer = (my + off) % N_DEV
            pl.semaphore_signal(
                barrier_sem, inc=1,
                device_id=(peer,), device_id_type=pl.DeviceIdType.MESH,
            )
        pl.semaphore_wait(barrier_sem, N_DEV - 1)

        x_dma.wait()
        xbf_ref[:, :] = xf32_ref[:, :].astype(jnp.bfloat16)

        sends = []
        for off in (1, 3, 2):
            dst = (my + off) % N_DEV
            rdma = pltpu.make_async_remote_copy(
                src_ref=xbf_ref.at[pl.ds(dst * BLK, BLK), :],
                dst_ref=comm_ref.at[my],
                send_sem=send_sems.at[off - 1],
                recv_sem=recv_sems.at[my],
                device_id=(dst,),
                device_id_type=pl.DeviceIdType.MESH,
            )
            rdma.start()
            sends.append(rdma)

        for j in range(N_DEV):
            w_dmas[j].wait()
            sl = pl.ds(j * BLK, BLK)
            wbf_ref[sl, :] = wf32_ref[sl, :].astype(jnp.bfloat16)

        acc = jnp.dot(
            xbf_ref[pl.ds(my * BLK, BLK), :],
            wbf_ref[pl.ds(my * BLK, BLK), :],
            preferred_element_type=jnp.float32,
        )

        for off in (1, 3, 2):
            src = (my + off) % N_DEV
            recv = pltpu.make_async_remote_copy(
                src_ref=comm_ref.at[src],
                dst_ref=comm_ref.at[src],
                send_sem=send_sems.at[0],
                recv_sem=recv_sems.at[src],
                device_id=(src,),
                device_id_type=pl.DeviceIdType.MESH,
            )
            recv.wait_recv()
            acc = acc + jnp.dot(
                comm_ref[src],
                wbf_ref[pl.ds(src * BLK, BLK), :],
                preferred_element_type=jnp.float32,
            )

        c = 0.7978845608028654
        o_dmas = []
        for q in range(N_DEV):
            sl = pl.ds(q * BLK, BLK)
            y = acc[:, q * BLK:(q + 1) * BLK]
            obuf_ref[:, sl] = 0.5 * y * (1.0 + jnp.tanh(c * (y + 0.044715 * y * y * y)))
            dma = pltpu.make_async_copy(
                obuf_ref.at[:, sl], out_hbm.at[:, sl], o_sems.at[q])
            dma.start()
            o_dmas.append(dma)

        for dma in o_dmas:
            dma.wait()
        for rdma in sends:
            rdma.wait_send()

    return pl.pallas_call(
        body,
        out_shape=jax.ShapeDtypeStruct((BLK, N), jnp.float32),
        in_specs=[
            pl.BlockSpec(memory_space=pltpu.MemorySpace.HBM),
            pl.BlockSpec(memory_space=pltpu.MemorySpace.HBM),
        ],
        out_specs=pl.BlockSpec(memory_space=pltpu.MemorySpace.HBM),
        scratch_shapes=[
            pltpu.VMEM((N_DEV * BLK, BLK), jnp.float32),
            pltpu.VMEM((N_DEV * BLK, BLK), jnp.bfloat16),
            pltpu.VMEM((N_DEV * BLK, N), jnp.float32),
            pltpu.VMEM((N_DEV * BLK, N), jnp.bfloat16),
            pltpu.VMEM((N_DEV, BLK, BLK), jnp.bfloat16),
            pltpu.VMEM((BLK, N), jnp.float32),
            pltpu.SemaphoreType.DMA,
            pltpu.SemaphoreType.DMA((N_DEV,)),
            pltpu.SemaphoreType.DMA((N_DEV,)),
            pltpu.SemaphoreType.DMA((N_DEV - 1,)),
            pltpu.SemaphoreType.DMA((N_DEV,)),
        ],
        compiler_params=pltpu.CompilerParams(
            collective_id=0, vmem_limit_bytes=64 * 1024 * 1024,
        ),
    )(x, w_mat)


# device time: 23369 ns/iter; 1.3267x vs baseline; 1.3267x over previous
import jax
import jax.numpy as jnp
from jax import lax
from jax.experimental import pallas as pl
from jax.experimental.pallas import tpu as pltpu

N_DEV = 4
BLK = 512
N = 2048


def kernel(x, w_mat):
    def body(x_ref, w_hbm, out_ref, xbf_ref, wf32_ref, comm_ref,
             w_sems, send_sems, recv_sems):
        my = lax.axis_index("i")

        barrier_sem = pltpu.get_barrier_semaphore()
        for off in range(1, N_DEV):
            peer = (my + off) % N_DEV
            pl.semaphore_signal(
                barrier_sem, inc=1,
                device_id=(peer,), device_id_type=pl.DeviceIdType.MESH,
            )
        pl.semaphore_wait(barrier_sem, N_DEV - 1)

        xbf_ref[:, :] = x_ref[:, :].astype(jnp.bfloat16)

        sends = []
        for off in (1, 3, 2):
            dst = (my + off) % N_DEV
            rdma = pltpu.make_async_remote_copy(
                src_ref=xbf_ref.at[pl.ds(dst * BLK, BLK), :],
                dst_ref=comm_ref.at[my],
                send_sem=send_sems.at[off - 1],
                recv_sem=recv_sems.at[my],
                device_id=(dst,),
                device_id_type=pl.DeviceIdType.MESH,
            )
            rdma.start()
            sends.append(rdma)

        w_dmas = []
        for off in (0, 1, 3, 2):
            j = (my + off) % N_DEV
            dma = pltpu.make_async_copy(
                w_hbm.at[pl.ds(j * BLK, BLK), :],
                wf32_ref.at[j],
                w_sems.at[j],
            )
            dma.start()
            w_dmas.append(dma)

        w_dmas[0].wait()
        acc = jnp.dot(
            xbf_ref[pl.ds(my * BLK, BLK), :],
            wf32_ref[my].astype(jnp.bfloat16),
            preferred_element_type=jnp.float32,
        )

        for i, off in enumerate((1, 3, 2)):
            src = (my + off) % N_DEV
            recv = pltpu.make_async_remote_copy(
                src_ref=comm_ref.at[src],
                dst_ref=comm_ref.at[src],
                send_sem=send_sems.at[0],
                recv_sem=recv_sems.at[src],
                device_id=(src,),
                device_id_type=pl.DeviceIdType.MESH,
            )
            recv.wait_recv()
            w_dmas[i + 1].wait()
            acc = acc + jnp.dot(
                comm_ref[src],
                wf32_ref[src].astype(jnp.bfloat16),
                preferred_element_type=jnp.float32,
            )

        c = 0.7978845608028654
        out_ref[:, :] = 0.5 * acc * (1.0 + jnp.tanh(c * (acc + 0.044715 * acc * acc * acc)))

        for rdma in sends:
            rdma.wait_send()

    return pl.pallas_call(
        body,
        out_shape=jax.ShapeDtypeStruct((BLK, N), jnp.float32),
        in_specs=[
            pl.BlockSpec(memory_space=pltpu.VMEM),
            pl.BlockSpec(memory_space=pltpu.MemorySpace.HBM),
        ],
        out_specs=pl.BlockSpec(memory_space=pltpu.VMEM),
        scratch_shapes=[
            pltpu.VMEM((N_DEV * BLK, BLK), jnp.bfloat16),
            pltpu.VMEM((N_DEV, BLK, N), jnp.float32),
            pltpu.VMEM((N_DEV, BLK, BLK), jnp.bfloat16),
            pltpu.SemaphoreType.DMA((N_DEV,)),
            pltpu.SemaphoreType.DMA((N_DEV - 1,)),
            pltpu.SemaphoreType.DMA((N_DEV,)),
        ],
        compiler_params=pltpu.CompilerParams(collective_id=0),
    )(x, w_mat)
